# baseline (speedup 1.0000x reference)
import functools

import jax
import jax.numpy as jnp
from jax import lax
from jax.experimental import pallas as pl
from jax.experimental.pallas import tpu as pltpu
from jax.experimental.pallas import tpu_sc as plsc

B = 16384
N_SPARSE = 26
N_DENSE = 13
N_COLS = N_SPARSE + N_DENSE
VOCAB = 1000000
L = 16


def _build_sc_kernel():
    info = plsc.get_sparse_core_info()
    nc, ns = info.num_cores, info.num_subcores
    nw = nc * ns
    rows_w = B // nw
    blocks_w = rows_w // L
    n_idx = N_SPARSE * rows_w
    idx_rows = n_idx // 128

    mesh = plsc.VectorSubcoreMesh(core_axis_name="c", subcore_axis_name="s")

    @functools.partial(
        pl.kernel,
        out_type=jax.ShapeDtypeStruct((B,), jnp.float32),
        mesh=mesh,
        scratch_types=[
            pltpu.VMEM((N_COLS, rows_w), jnp.float32),
            pltpu.VMEM((idx_rows, 128), jnp.int32),
            pltpu.VMEM((idx_rows, 128), jnp.float32),
            pltpu.VMEM((N_DENSE * L,), jnp.float32),
            pltpu.VMEM((rows_w,), jnp.float32),
            pltpu.SemaphoreType.DMA,
        ],
    )
    def sc_kernel(xt_hbm, table_hbm, wb_hbm, out_hbm, xt_v, idx_v, gath_v,
                  wb_v, out_v, sem):
        wid = lax.axis_index("s") * nc + lax.axis_index("c")
        base_row = wid * rows_w

        pltpu.sync_copy(xt_hbm.at[:, pl.ds(base_row, rows_w)], xt_v)
        pltpu.sync_copy(wb_hbm, wb_v)

        def idx_body(t, _):
            f = t // blocks_w
            blk = t % blocks_w
            xv = xt_v[f, pl.ds(blk * L, L)]
            iv = xv.astype(jnp.int32) + f * VOCAB
            idx_v[t // 8, pl.ds((t % 8) * L, L)] = iv
            return 0

        lax.fori_loop(0, N_SPARSE * blocks_w, idx_body, 0)

        def fire(j, _):
            pltpu.async_copy(table_hbm.at[idx_v.at[j]], gath_v.at[j], sem)
            return 0

        lax.fori_loop(0, idx_rows, fire, 0)

        def drain(j, _):
            pltpu.make_async_copy(table_hbm.at[idx_v.at[0]], gath_v.at[0],
                                  sem).wait()
            return 0

        lax.fori_loop(0, idx_rows, drain, 0)

        def blk_body(blk, _):
            def red_body(f, acc):
                t = f * blocks_w + blk
                g = gath_v[t // 8, pl.ds((t % 8) * L, L)]
                return acc + g

            acc = lax.fori_loop(0, N_SPARSE, red_body,
                                jnp.zeros((L,), jnp.float32))

            def dense_body(d, dacc):
                xv = xt_v[N_SPARSE + d, pl.ds(blk * L, L)]
                wv = wb_v[pl.ds(d * L, L)]
                return dacc + xv * wv

            acc = lax.fori_loop(0, N_DENSE, dense_body, acc)
            out_v[pl.ds(blk * L, L)] = 1.0 / (1.0 + jnp.exp(-acc))
            return 0

        lax.fori_loop(0, blocks_w, blk_body, 0)

        pltpu.sync_copy(out_v, out_hbm.at[pl.ds(base_row, rows_w)])

    return sc_kernel


def kernel(X, emb_tables, dense_weight):
    xt = X.reshape(N_COLS, B)
    table_flat = emb_tables.reshape(-1)
    w_bcast = jnp.repeat(dense_weight.reshape(-1), L)
    out = _build_sc_kernel()(xt, table_flat, w_bcast)
    return out[:, None]


# speedup vs baseline: 1.0348x; 1.0348x over previous
import functools

import jax
import jax.numpy as jnp
from jax import lax
from jax.experimental import pallas as pl
from jax.experimental.pallas import tpu as pltpu
from jax.experimental.pallas import tpu_sc as plsc

B = 16384
N_SPARSE = 26
N_DENSE = 13
N_COLS = N_SPARSE + N_DENSE
VOCAB = 1000000
L = 16


def _build_sc_kernel():
    info = plsc.get_sparse_core_info()
    nc, ns = info.num_cores, info.num_subcores
    nw = nc * ns
    rows_w = B // nw
    blocks_w = rows_w // L
    n_idx = N_SPARSE * rows_w
    idx_rows = n_idx // 128

    mesh = plsc.VectorSubcoreMesh(core_axis_name="c", subcore_axis_name="s")

    @functools.partial(
        pl.kernel,
        out_type=jax.ShapeDtypeStruct((B,), jnp.float32),
        mesh=mesh,
        scratch_types=[
            pltpu.VMEM((N_COLS, rows_w), jnp.float32),
            pltpu.VMEM((n_idx,), jnp.int32),
            pltpu.VMEM((n_idx,), jnp.float32),
            pltpu.VMEM((N_DENSE * L,), jnp.float32),
            pltpu.VMEM((rows_w,), jnp.float32),
            pltpu.SemaphoreType.DMA,
        ],
    )
    def sc_kernel(xt_hbm, table_hbm, wb_hbm, out_hbm, xt_v, idx_v, gath_v,
                  wb_v, out_v, sem):
        wid = lax.axis_index("s") * nc + lax.axis_index("c")
        base_row = wid * rows_w

        pltpu.sync_copy(xt_hbm.at[:, pl.ds(base_row, rows_w)], xt_v)
        pltpu.sync_copy(wb_hbm, wb_v)

        def idx_body(t, _):
            f = t // blocks_w
            blk = t % blocks_w
            xv = xt_v[f, pl.ds(blk * L, L)]
            iv = xv.astype(jnp.int32) + f * VOCAB
            idx_v[pl.ds(t * L, L)] = iv
            return 0

        lax.fori_loop(0, N_SPARSE * blocks_w, idx_body, 0)

        pltpu.async_copy(table_hbm.at[idx_v], gath_v, sem).wait()

        def blk_body(blk, _):
            def red_body(f, acc):
                g = gath_v[pl.ds(f * rows_w + blk * L, L)]
                return acc + g

            acc = lax.fori_loop(0, N_SPARSE, red_body,
                                jnp.zeros((L,), jnp.float32))

            def dense_body(d, dacc):
                xv = xt_v[N_SPARSE + d, pl.ds(blk * L, L)]
                wv = wb_v[pl.ds(d * L, L)]
                return dacc + xv * wv

            acc = lax.fori_loop(0, N_DENSE, dense_body, acc)
            out_v[pl.ds(blk * L, L)] = 1.0 / (1.0 + jnp.exp(-acc))
            return 0

        lax.fori_loop(0, blocks_w, blk_body, 0)

        pltpu.sync_copy(out_v, out_hbm.at[pl.ds(base_row, rows_w)])

    return sc_kernel


def kernel(X, emb_tables, dense_weight):
    xt = X.T
    table_flat = emb_tables.reshape(-1)
    w_bcast = jnp.repeat(dense_weight.reshape(-1), L)
    out = _build_sc_kernel()(xt, table_flat, w_bcast)
    return out[:, None]


# speedup vs baseline: 1.8535x; 1.7911x over previous
import functools

import jax
import jax.numpy as jnp
from jax import lax
from jax.experimental import pallas as pl
from jax.experimental.pallas import tpu as pltpu
from jax.experimental.pallas import tpu_sc as plsc

B = 16384
N_SPARSE = 26
N_DENSE = 13
N_COLS = N_SPARSE + N_DENSE
VOCAB = 1000000
VOCAB_PAD = 1000064
L = 16


def _build_sc_kernel():
    info = plsc.get_sparse_core_info()
    nc, ns = info.num_cores, info.num_subcores
    nw = nc * ns
    rows_w = B // nw
    blocks_w = rows_w // L
    n_idx = N_SPARSE * rows_w
    idx_rows = n_idx // 128

    mesh = plsc.VectorSubcoreMesh(core_axis_name="c", subcore_axis_name="s")

    @functools.partial(
        pl.kernel,
        out_type=jax.ShapeDtypeStruct((B,), jnp.float32),
        mesh=mesh,
        scratch_types=[
            pltpu.VMEM((N_COLS, rows_w), jnp.float32),
            pltpu.VMEM((n_idx,), jnp.int32),
            pltpu.VMEM((n_idx,), jnp.float32),
            pltpu.VMEM((N_DENSE * L,), jnp.float32),
            pltpu.VMEM((rows_w,), jnp.float32),
            pltpu.SemaphoreType.DMA,
        ],
    )
    def sc_kernel(xt_hbm, table_hbm, wb_hbm, out_hbm, xt_v, idx_v, gath_v,
                  wb_v, out_v, sem):
        wid = lax.axis_index("s") * nc + lax.axis_index("c")
        base_row = wid * rows_w

        pltpu.sync_copy(xt_hbm.at[:, pl.ds(base_row, rows_w)], xt_v)
        pltpu.sync_copy(wb_hbm, wb_v)

        def idx_body(t, _):
            f = t // blocks_w
            blk = t % blocks_w
            xv = xt_v[f, pl.ds(blk * L, L)]
            iv = xv.astype(jnp.int32) + f * VOCAB
            idx_v[pl.ds(t * L, L)] = iv
            return 0

        lax.fori_loop(0, N_SPARSE * blocks_w, idx_body, 0)

        pltpu.async_copy(table_hbm.at[idx_v], gath_v, sem).wait()

        def blk_body(blk, _):
            def red_body(f, acc):
                g = gath_v[pl.ds(f * rows_w + blk * L, L)]
                return acc + g

            acc = lax.fori_loop(0, N_SPARSE, red_body,
                                jnp.zeros((L,), jnp.float32))

            def dense_body(d, dacc):
                xv = xt_v[N_SPARSE + d, pl.ds(blk * L, L)]
                wv = wb_v[pl.ds(d * L, L)]
                return dacc + xv * wv

            acc = lax.fori_loop(0, N_DENSE, dense_body, acc)
            out_v[pl.ds(blk * L, L)] = 1.0 / (1.0 + jnp.exp(-acc))
            return 0

        lax.fori_loop(0, blocks_w, blk_body, 0)

        pltpu.sync_copy(out_v, out_hbm.at[pl.ds(base_row, rows_w)])

    return sc_kernel


def kernel(X, emb_tables, dense_weight):
    xt = X.T
    table_flat = jnp.concatenate(
        [emb_tables[f, :, 0] for f in range(N_SPARSE)])
    w_bcast = jnp.repeat(dense_weight.reshape(-1), L)
    out = _build_sc_kernel()(xt, table_flat, w_bcast)
    return out[:, None]
